# Initial kernel scaffold; baseline (speedup 1.0000x reference)
import jax
import jax.numpy as jnp
from jax.experimental import pallas as pl


def kernel(x, edge_index, batch, edge_weight, Wrel, Wroot, bconv, gamma, beta, run_mean, run_var, W1, b1, W2, b2):
    raise NotImplementedError("write your pallas kernel here")


# speedup vs baseline: 3.7706x; 3.7706x over previous
import functools

import jax
import jax.numpy as jnp
from jax import lax
from jax.experimental import pallas as pl
from jax.experimental.pallas import tpu as pltpu
from jax.experimental.pallas import tpu_sc as plsc

N = 10000
E = 320000
D = 128
G = 64
C = 10
L = 4
EPS = 1e-5

NC = 2
NS = 16
EW = E // (NC * NS)
B = 80
NCHUNK = EW // B
NP = 10240
ROWS_PER_TILE = NP // NS
ZROWS = 128


@functools.partial(
    pl.kernel,
    out_type=jax.ShapeDtypeStruct((NC, NP, D), jnp.float32),
    mesh=plsc.VectorSubcoreMesh(core_axis_name="c", subcore_axis_name="s"),
    scratch_types=[
        pltpu.VMEM((B,), jnp.int32),
        pltpu.VMEM((B,), jnp.int32),
        pltpu.VMEM((B,), jnp.float32),
        pltpu.VMEM((B, D), jnp.float32),
        pltpu.VMEM((ZROWS, D), jnp.float32),
        pltpu.VMEM_SHARED((NP, D), jnp.float32),
        pltpu.SemaphoreType.DMA,
    ],
)
def _sc_agg(h_hbm, src_hbm, dst_hbm, ew_hbm, out_hbm,
            src_v, dst_v, w_v, rows_v, zbuf_v, acc_ref, sem):
    c = lax.axis_index("c")
    s = lax.axis_index("s")

    def zrow(r, _):
        for j in range(D // 16):
            zbuf_v[r, pl.ds(j * 16, 16)] = jnp.zeros((16,), jnp.float32)
        return 0
    lax.fori_loop(0, ZROWS, zrow, 0)

    def zcopy(p, _):
        pltpu.sync_copy(
            zbuf_v, acc_ref.at[pl.ds(s * ROWS_PER_TILE + p * ZROWS, ZROWS)])
        return 0
    lax.fori_loop(0, ROWS_PER_TILE // ZROWS, zcopy, 0)
    plsc.subcore_barrier()

    ebase = c * (E // NC) + s * EW

    def chunk(k, _):
        base = ebase + k * B
        pltpu.sync_copy(src_hbm.at[pl.ds(base, B)], src_v)
        pltpu.sync_copy(dst_hbm.at[pl.ds(base, B)], dst_v)
        pltpu.sync_copy(ew_hbm.at[pl.ds(base, B)], w_v)
        pltpu.async_copy(h_hbm.at[src_v], rows_v, sem).wait()

        def rowgrp(q, _):
            r0 = q * 16
            wchunk = w_v[pl.ds(r0, 16)]
            for t in range(16):
                wvec = lax.gather(
                    wchunk, jnp.full((16, 1), t, jnp.int32),
                    lax.GatherDimensionNumbers(
                        offset_dims=(), collapsed_slice_dims=(0,),
                        start_index_map=(0,)),
                    (1,), mode=lax.GatherScatterMode.PROMISE_IN_BOUNDS)
                r = r0 + t
                for j in range(D // 16):
                    rows_v[r, pl.ds(j * 16, 16)] = (
                        rows_v[r, pl.ds(j * 16, 16)] * wvec)
            return 0
        lax.fori_loop(0, B // 16, rowgrp, 0)

        pltpu.sync_copy(rows_v, acc_ref.at[dst_v], add=True)
        return 0
    lax.fori_loop(0, NCHUNK, chunk, 0)

    plsc.subcore_barrier()

    def wb(p, _):
        r0 = s * ROWS_PER_TILE + p * ZROWS
        pltpu.sync_copy(acc_ref.at[pl.ds(r0, ZROWS)], zbuf_v)
        pltpu.sync_copy(zbuf_v, out_hbm.at[c, pl.ds(r0, ZROWS)])
        return 0
    lax.fori_loop(0, ROWS_PER_TILE // ZROWS, wb, 0)


RBLK = 1000


def _tc_layer_body(agg_ref, h_ref, wrel_ref, wroot_ref, bias_ref, out_ref):
    agg = agg_ref[0] + agg_ref[1]
    z = (jnp.dot(agg, wrel_ref[...], preferred_element_type=jnp.float32)
         + jnp.dot(h_ref[...], wroot_ref[...], preferred_element_type=jnp.float32)
         + bias_ref[...])
    out_ref[...] = jnp.maximum(z, 0.0)


def _tc_layer(agg2, h, wrel, wroot, bias):
    return pl.pallas_call(
        _tc_layer_body,
        grid=(N // RBLK,),
        in_specs=[
            pl.BlockSpec((NC, RBLK, D), lambda i: (0, i, 0)),
            pl.BlockSpec((RBLK, D), lambda i: (i, 0)),
            pl.BlockSpec((D, D), lambda i: (0, 0)),
            pl.BlockSpec((D, D), lambda i: (0, 0)),
            pl.BlockSpec((1, D), lambda i: (0, 0)),
        ],
        out_specs=pl.BlockSpec((RBLK, D), lambda i: (i, 0)),
        out_shape=jax.ShapeDtypeStruct((N, D), jnp.float32),
    )(agg2, h, wrel, wroot, bias)


def _tc_pool_head_body(h_ref, batch_ref, w1_ref, b1_ref, w2_ref, b2_ref,
                       out_ref, pool_acc):
    i = pl.program_id(0)

    @pl.when(i == 0)
    def _():
        pool_acc[...] = jnp.zeros_like(pool_acc)

    b = batch_ref[0, 0, :]
    gid = lax.broadcasted_iota(jnp.int32, (RBLK, G), 1)
    onehot = jnp.where(b[:, None] == gid, 1.0, 0.0).astype(jnp.float32)
    pool_acc[...] += lax.dot_general(
        onehot, h_ref[...], (((0,), (0,)), ((), ())),
        preferred_element_type=jnp.float32)

    @pl.when(i == pl.num_programs(0) - 1)
    def _():
        t = jnp.maximum(
            jnp.dot(pool_acc[...], w1_ref[...],
                    preferred_element_type=jnp.float32) + b1_ref[...], 0.0)
        logits = jnp.dot(t, w2_ref[...],
                         preferred_element_type=jnp.float32) + b2_ref[...]
        m = jnp.max(logits, axis=-1, keepdims=True)
        lse = jnp.log(jnp.sum(jnp.exp(logits - m), axis=-1, keepdims=True))
        out_ref[...] = logits - m - lse


def _tc_pool_head(h, batch3, w1, b1, w2p, b2p):
    return pl.pallas_call(
        _tc_pool_head_body,
        grid=(N // RBLK,),
        in_specs=[
            pl.BlockSpec((RBLK, D), lambda i: (i, 0)),
            pl.BlockSpec((1, 1, RBLK), lambda i: (i, 0, 0)),
            pl.BlockSpec((D, D), lambda i: (0, 0)),
            pl.BlockSpec((1, D), lambda i: (0, 0)),
            pl.BlockSpec((D, D), lambda i: (0, 0)),
            pl.BlockSpec((1, D), lambda i: (0, 0)),
        ],
        out_specs=pl.BlockSpec((G, D), lambda i: (0, 0)),
        out_shape=jax.ShapeDtypeStruct((G, D), jnp.float32),
        scratch_shapes=[pltpu.VMEM((G, D), jnp.float32)],
    )(h, batch3, w1, b1, w2p, b2p)


def kernel(x, edge_index, batch, edge_weight, Wrel, Wroot, bconv, gamma,
           beta, run_mean, run_var, W1, b1, W2, b2):
    src = edge_index[0]
    dst = edge_index[1]

    scale = gamma / jnp.sqrt(run_var + EPS)
    wrel = Wrel * scale[:, None, :]
    wroot = Wroot * scale[:, None, :]
    bias = (bconv - run_mean) * scale + beta

    w2p = jnp.zeros((D, D), jnp.float32).at[:, :C].set(W2)
    b2p = jnp.full((D,), -1e30, jnp.float32).at[:C].set(b2)

    h = x
    for i in range(L):
        parts = _sc_agg(h, src, dst, edge_weight)
        h = _tc_layer(parts, h, wrel[i], wroot[i], bias[i].reshape(1, D))

    batch3 = batch.reshape(N // RBLK, 1, RBLK)
    out = _tc_pool_head(h, batch3, W1, b1.reshape(1, D), w2p,
                        b2p.reshape(1, D))
    return out[:, :C]


# speedup vs baseline: 11.3875x; 3.0201x over previous
import functools

import jax
import jax.numpy as jnp
from jax import lax
from jax.experimental import pallas as pl
from jax.experimental.pallas import tpu as pltpu
from jax.experimental.pallas import tpu_sc as plsc

N = 10000
E = 320000
D = 128
G = 64
C = 10
L = 4
EPS = 1e-5

NC = 2
NS = 16
EW = E // (NC * NS)
B = 80
NCHUNK = EW // B
NP = 10240
ROWS_PER_TILE = NP // NS


@functools.partial(
    pl.kernel,
    out_type=jax.ShapeDtypeStruct((NC, NP, D), jnp.float32),
    mesh=plsc.VectorSubcoreMesh(core_axis_name="c", subcore_axis_name="s"),
    scratch_types=[
        pltpu.VMEM((EW,), jnp.int32),
        pltpu.VMEM((3, B, D), jnp.float32),
        pltpu.VMEM((3, B), jnp.int32),
        pltpu.VMEM((3, B), jnp.float32),
        pltpu.VMEM_SHARED((NP, D), jnp.float32),
        [pltpu.SemaphoreType.DMA] * 3,
        [pltpu.SemaphoreType.DMA] * 3,
    ],
)
def _sc_agg(h_hbm, src_hbm, dst_hbm, ew_hbm, out_hbm,
            src_all, rows3, didx3, w3, acc_ref, gsems, ssems):
    c = lax.axis_index("c")
    s = lax.axis_index("s")

    def zrow(r, _):
        for j in range(D // 16):
            rows3[0, r, pl.ds(j * 16, 16)] = jnp.zeros((16,), jnp.float32)
        return 0
    lax.fori_loop(0, B, zrow, 0)

    def zcopy(p, _):
        pltpu.sync_copy(
            rows3.at[0], acc_ref.at[pl.ds(s * ROWS_PER_TILE + p * B, B)])
        return 0
    lax.fori_loop(0, ROWS_PER_TILE // B, zcopy, 0)

    ebase = (c * NS + s) * EW
    pltpu.sync_copy(src_hbm.at[pl.ds(ebase, EW)], src_all)
    plsc.subcore_barrier()

    def fire_gather(k, b):
        pltpu.async_copy(dst_hbm.at[pl.ds(ebase + k * B, B)], didx3.at[b],
                         gsems[b])
        pltpu.async_copy(ew_hbm.at[pl.ds(ebase + k * B, B)], w3.at[b],
                         gsems[b])
        pltpu.async_copy(h_hbm.at[src_all.at[pl.ds(k * B, B)]],
                         rows3.at[b], gsems[b])

    def wait_gather(k, b):
        pltpu.make_async_copy(dst_hbm.at[pl.ds(ebase + k * B, B)],
                              didx3.at[b], gsems[b]).wait()
        pltpu.make_async_copy(ew_hbm.at[pl.ds(ebase + k * B, B)], w3.at[b],
                              gsems[b]).wait()
        pltpu.make_async_copy(h_hbm.at[src_all.at[pl.ds(k * B, B)]],
                              rows3.at[b], gsems[b]).wait()

    def fire_scatter(b):
        pltpu.async_copy(rows3.at[b], acc_ref.at[didx3.at[b]], ssems[b],
                         add=True)

    def wait_scatter(b):
        pltpu.make_async_copy(rows3.at[b], acc_ref.at[didx3.at[b]],
                              ssems[b]).wait()

    def process(k, b):
        def rowgrp(q, _):
            r0 = q * 16
            wchunk = w3[b, pl.ds(r0, 16)]
            for t in range(16):
                wvec = lax.gather(
                    wchunk, jnp.full((16, 1), t, jnp.int32),
                    lax.GatherDimensionNumbers(
                        offset_dims=(), collapsed_slice_dims=(0,),
                        start_index_map=(0,)),
                    (1,), mode=lax.GatherScatterMode.PROMISE_IN_BOUNDS)
                r = r0 + t
                for j in range(D // 16):
                    rows3[b, r, pl.ds(j * 16, 16)] = (
                        rows3[b, r, pl.ds(j * 16, 16)] * wvec)
            return 0
        lax.fori_loop(0, B // 16, rowgrp, 0)

    fire_gather(0, 0)
    fire_gather(1, 1)

    wait_gather(0, 0)
    process(0, 0)
    fire_gather(2, 2)
    fire_scatter(0)
    wait_gather(1, 1)
    process(1, 1)
    wait_scatter(0)
    fire_gather(3, 0)
    fire_scatter(1)
    wait_gather(2, 2)
    process(2, 2)
    wait_scatter(1)
    fire_gather(4, 1)
    fire_scatter(2)

    def triple(p, _):
        k0 = 3 * p
        for t in range(3):
            b = t
            k = k0 + t
            wait_gather(k, b)
            process(k, b)
            bn = (t + 2) % 3
            wait_scatter(bn)
            fire_gather(k + 2, bn)
            fire_scatter(b)
        return 0
    lax.fori_loop(1, (NCHUNK - 2) // 3, triple, 0)

    wait_gather(NCHUNK - 2, 0)
    process(NCHUNK - 2, 0)
    wait_scatter(2)
    fire_scatter(0)
    wait_gather(NCHUNK - 1, 1)
    process(NCHUNK - 1, 1)
    fire_scatter(1)
    wait_scatter(0)
    wait_scatter(1)

    plsc.subcore_barrier()

    def wb(p, _):
        r0 = s * ROWS_PER_TILE + p * B
        pltpu.sync_copy(acc_ref.at[pl.ds(r0, B)], rows3.at[0])
        pltpu.sync_copy(rows3.at[0], out_hbm.at[c, pl.ds(r0, B)])
        return 0
    lax.fori_loop(0, ROWS_PER_TILE // B, wb, 0)


RBLK = 1000


def _tc_layer_body(agg_ref, h_ref, wrel_ref, wroot_ref, bias_ref, out_ref):
    agg = agg_ref[0] + agg_ref[1]
    z = (jnp.dot(agg, wrel_ref[...], preferred_element_type=jnp.float32)
         + jnp.dot(h_ref[...], wroot_ref[...], preferred_element_type=jnp.float32)
         + bias_ref[...])
    out_ref[...] = jnp.maximum(z, 0.0)


def _tc_layer(agg2, h, wrel, wroot, bias):
    return pl.pallas_call(
        _tc_layer_body,
        grid=(N // RBLK,),
        in_specs=[
            pl.BlockSpec((NC, RBLK, D), lambda i: (0, i, 0)),
            pl.BlockSpec((RBLK, D), lambda i: (i, 0)),
            pl.BlockSpec((D, D), lambda i: (0, 0)),
            pl.BlockSpec((D, D), lambda i: (0, 0)),
            pl.BlockSpec((1, D), lambda i: (0, 0)),
        ],
        out_specs=pl.BlockSpec((RBLK, D), lambda i: (i, 0)),
        out_shape=jax.ShapeDtypeStruct((N, D), jnp.float32),
    )(agg2, h, wrel, wroot, bias)


def _tc_pool_head_body(h_ref, batch_ref, w1_ref, b1_ref, w2_ref, b2_ref,
                       out_ref, pool_acc):
    i = pl.program_id(0)

    @pl.when(i == 0)
    def _():
        pool_acc[...] = jnp.zeros_like(pool_acc)

    b = batch_ref[0, 0, :]
    gid = lax.broadcasted_iota(jnp.int32, (RBLK, G), 1)
    onehot = jnp.where(b[:, None] == gid, 1.0, 0.0).astype(jnp.float32)
    pool_acc[...] += lax.dot_general(
        onehot, h_ref[...], (((0,), (0,)), ((), ())),
        preferred_element_type=jnp.float32)

    @pl.when(i == pl.num_programs(0) - 1)
    def _():
        t = jnp.maximum(
            jnp.dot(pool_acc[...], w1_ref[...],
                    preferred_element_type=jnp.float32) + b1_ref[...], 0.0)
        logits = jnp.dot(t, w2_ref[...],
                         preferred_element_type=jnp.float32) + b2_ref[...]
        m = jnp.max(logits, axis=-1, keepdims=True)
        lse = jnp.log(jnp.sum(jnp.exp(logits - m), axis=-1, keepdims=True))
        out_ref[...] = logits - m - lse


def _tc_pool_head(h, batch3, w1, b1, w2p, b2p):
    return pl.pallas_call(
        _tc_pool_head_body,
        grid=(N // RBLK,),
        in_specs=[
            pl.BlockSpec((RBLK, D), lambda i: (i, 0)),
            pl.BlockSpec((1, 1, RBLK), lambda i: (i, 0, 0)),
            pl.BlockSpec((D, D), lambda i: (0, 0)),
            pl.BlockSpec((1, D), lambda i: (0, 0)),
            pl.BlockSpec((D, D), lambda i: (0, 0)),
            pl.BlockSpec((1, D), lambda i: (0, 0)),
        ],
        out_specs=pl.BlockSpec((G, D), lambda i: (0, 0)),
        out_shape=jax.ShapeDtypeStruct((G, D), jnp.float32),
        scratch_shapes=[pltpu.VMEM((G, D), jnp.float32)],
    )(h, batch3, w1, b1, w2p, b2p)


def kernel(x, edge_index, batch, edge_weight, Wrel, Wroot, bconv, gamma,
           beta, run_mean, run_var, W1, b1, W2, b2):
    src = edge_index[0]
    dst = edge_index[1]

    scale = gamma / jnp.sqrt(run_var + EPS)
    wrel = Wrel * scale[:, None, :]
    wroot = Wroot * scale[:, None, :]
    bias = (bconv - run_mean) * scale + beta

    w2p = jnp.zeros((D, D), jnp.float32).at[:, :C].set(W2)
    b2p = jnp.full((D,), -1e30, jnp.float32).at[:C].set(b2)

    h = x
    for i in range(L):
        parts = _sc_agg(h, src, dst, edge_weight)
        h = _tc_layer(parts, h, wrel[i], wroot[i], bias[i].reshape(1, D))

    batch3 = batch.reshape(N // RBLK, 1, RBLK)
    out = _tc_pool_head(h, batch3, W1, b1.reshape(1, D), w2p,
                        b2p.reshape(1, D))
    return out[:, :C]
